# Initial kernel scaffold; baseline (speedup 1.0000x reference)
import jax
import jax.numpy as jnp
from jax.experimental import pallas as pl


def kernel(x, weight, weight_rc):
    raise NotImplementedError("write your pallas kernel here")


# speedup vs baseline: 80.2839x; 80.2839x over previous
import functools

import jax
import jax.numpy as jnp
from jax import lax
from jax.experimental import pallas as pl
from jax.experimental.pallas import tpu as pltpu
from jax.experimental.pallas import tpu_sc as plsc

NUM_CORES = 2
NUM_SUBCORES = 16
LANES = 16
NW = NUM_CORES * NUM_SUBCORES

B = 1024
L = 4096
E = 4
V = 5

B_PER_W = B // NW
CHUNKS = L // LANES

_mesh = plsc.VectorSubcoreMesh(core_axis_name="c", subcore_axis_name="s")


@functools.partial(
    pl.kernel,
    out_type=jax.ShapeDtypeStruct((2 * B, E, L), jnp.float32),
    mesh=_mesh,
    compiler_params=pltpu.CompilerParams(needs_layout_passes=False),
    scratch_types=[
        pltpu.VMEM((64,), jnp.float32),
        pltpu.VMEM((L,), jnp.int32),
        pltpu.VMEM((E, L), jnp.float32),
        pltpu.VMEM((E, L), jnp.float32),
    ],
)
def _emb_kernel(tbl_hbm, x_hbm, out_hbm, tbl_v, x_v, fwd_v, rc_v):
    wid = lax.axis_index("s") * NUM_CORES + lax.axis_index("c")
    pltpu.sync_copy(tbl_hbm, tbl_v)
    base = wid * B_PER_W

    def body_b(i, carry):
        b = base + i
        pltpu.sync_copy(x_hbm.at[b], x_v)

        def body_c(c, carry_c):
            xv = x_v[pl.ds(c * LANES, LANES)]
            xr = lax.rev(xv, (0,))
            for e in range(E):
                f = plsc.load_gather(tbl_v, [xv + (e * V)])
                fwd_v[e, pl.ds(c * LANES, LANES)] = f
                r = plsc.load_gather(tbl_v, [xr + (E * V + e * V)])
                rc_v[e, pl.ds(L - LANES - c * LANES, LANES)] = r
            return carry_c

        lax.fori_loop(0, CHUNKS, body_c, 0)
        pltpu.sync_copy(fwd_v, out_hbm.at[b])
        pltpu.sync_copy(rc_v, out_hbm.at[B + b])
        return carry

    lax.fori_loop(0, B_PER_W, body_b, 0)


def kernel(x, weight, weight_rc):
    tbl = jnp.concatenate([
        weight.T.reshape(-1),
        weight_rc.T.reshape(-1),
        jnp.zeros((64 - 2 * E * V,), jnp.float32),
    ])
    return _emb_kernel(tbl, x)


# speedup vs baseline: 98.4693x; 1.2265x over previous
import functools

import jax
import jax.numpy as jnp
from jax import lax
from jax.experimental import pallas as pl
from jax.experimental.pallas import tpu as pltpu
from jax.experimental.pallas import tpu_sc as plsc

NUM_CORES = 2
NUM_SUBCORES = 16
LANES = 16
NW = NUM_CORES * NUM_SUBCORES

B = 1024
L = 4096
E = 4
V = 5

B_PER_W = B // NW
CHUNKS = L // LANES

_mesh = plsc.VectorSubcoreMesh(core_axis_name="c", subcore_axis_name="s")


@functools.partial(
    pl.kernel,
    out_type=jax.ShapeDtypeStruct((2 * B, E, L), jnp.float32),
    mesh=_mesh,
    compiler_params=pltpu.CompilerParams(needs_layout_passes=False),
    scratch_types=[
        pltpu.VMEM((64,), jnp.float32),
        pltpu.VMEM((2, L), jnp.int32),
        pltpu.VMEM((2, E, L), jnp.float32),
        pltpu.VMEM((2, E, L), jnp.float32),
        pltpu.SemaphoreType.DMA,
        pltpu.SemaphoreType.DMA,
        pltpu.SemaphoreType.DMA,
        pltpu.SemaphoreType.DMA,
        pltpu.SemaphoreType.DMA,
        pltpu.SemaphoreType.DMA,
    ],
)
def _emb_kernel(tbl_hbm, x_hbm, out_hbm, tbl_v, x_v, fwd_v, rc_v,
                sx0, sx1, sf0, sf1, sr0, sr1):
    wid = lax.axis_index("s") * NUM_CORES + lax.axis_index("c")
    pltpu.sync_copy(tbl_hbm, tbl_v)
    base = wid * B_PER_W
    sx = (sx0, sx1)
    sf = (sf0, sf1)
    sr = (sr0, sr1)

    pltpu.async_copy(x_hbm.at[base], x_v.at[0], sx[0])

    def body_i2(i2, carry):
        for s in (0, 1):
            i = i2 * 2 + s
            b = base + i
            nxt = 1 - s

            if s == 0:
                pltpu.async_copy(x_hbm.at[b + 1], x_v.at[nxt], sx[nxt])
            else:
                @pl.when(i + 1 < B_PER_W)
                def _():
                    pltpu.async_copy(x_hbm.at[b + 1], x_v.at[nxt], sx[nxt])

            pltpu.make_async_copy(x_hbm.at[b], x_v.at[s], sx[s]).wait()

            @pl.when(i2 > 0)
            def _():
                pltpu.make_async_copy(fwd_v.at[s], out_hbm.at[b - 2],
                                      sf[s]).wait()
                pltpu.make_async_copy(rc_v.at[s], out_hbm.at[B + b - 2],
                                      sr[s]).wait()

            def body_c(c, carry_c):
                xv = x_v[s, pl.ds(c * LANES, LANES)]
                xr = lax.rev(xv, (0,))
                for e in range(E):
                    f = plsc.load_gather(tbl_v, [xv + (e * V)])
                    fwd_v[s, e, pl.ds(c * LANES, LANES)] = f
                    r = plsc.load_gather(tbl_v, [xr + (E * V + e * V)])
                    rc_v[s, e, pl.ds(L - LANES - c * LANES, LANES)] = r
                return carry_c

            lax.fori_loop(0, CHUNKS, body_c, 0)

            pltpu.async_copy(fwd_v.at[s], out_hbm.at[b], sf[s])
            pltpu.async_copy(rc_v.at[s], out_hbm.at[B + b], sr[s])
        return carry

    lax.fori_loop(0, B_PER_W // 2, body_i2, 0)

    last = base + B_PER_W - 2
    pltpu.make_async_copy(fwd_v.at[0], out_hbm.at[last], sf[0]).wait()
    pltpu.make_async_copy(rc_v.at[0], out_hbm.at[B + last], sr[0]).wait()
    pltpu.make_async_copy(fwd_v.at[1], out_hbm.at[last + 1], sf[1]).wait()
    pltpu.make_async_copy(rc_v.at[1], out_hbm.at[B + last + 1], sr[1]).wait()


def kernel(x, weight, weight_rc):
    tbl = jnp.concatenate([
        weight.T.reshape(-1),
        weight_rc.T.reshape(-1),
        jnp.zeros((64 - 2 * E * V,), jnp.float32),
    ])
    return _emb_kernel(tbl, x)


# speedup vs baseline: 415.3043x; 4.2176x over previous
import functools

import jax
import jax.numpy as jnp
from jax import lax
from jax.experimental import pallas as pl
from jax.experimental.pallas import tpu as pltpu
from jax.experimental.pallas import tpu_sc as plsc

NUM_CORES = 2
NUM_SUBCORES = 16
LANES = 16
NW = NUM_CORES * NUM_SUBCORES

B = 1024
L = 4096
E = 4
V = 5

B_PER_W = B // NW
CHUNKS = L // LANES

_mesh = plsc.VectorSubcoreMesh(core_axis_name="c", subcore_axis_name="s")


@functools.partial(
    pl.kernel,
    out_type=jax.ShapeDtypeStruct((2 * B, E, L), jnp.float32),
    mesh=_mesh,
    compiler_params=pltpu.CompilerParams(needs_layout_passes=False),
    scratch_types=[
        pltpu.VMEM((64,), jnp.float32),
        pltpu.VMEM((2, L), jnp.int32),
        pltpu.VMEM((2, E, L), jnp.float32),
        pltpu.VMEM((2, E, L), jnp.float32),
        pltpu.SemaphoreType.DMA,
        pltpu.SemaphoreType.DMA,
        pltpu.SemaphoreType.DMA,
        pltpu.SemaphoreType.DMA,
        pltpu.SemaphoreType.DMA,
        pltpu.SemaphoreType.DMA,
    ],
)
def _emb_kernel(tbl_hbm, x_hbm, out_hbm, tbl_v, x_v, fwd_v, rc_v,
                sx0, sx1, sf0, sf1, sr0, sr1):
    wid = lax.axis_index("s") * NUM_CORES + lax.axis_index("c")
    pltpu.sync_copy(tbl_hbm, tbl_v)
    base = wid * B_PER_W
    sx = (sx0, sx1)
    sf = (sf0, sf1)
    sr = (sr0, sr1)

    pltpu.async_copy(x_hbm.at[base], x_v.at[0], sx[0])

    def body_i2(i2, carry):
        for s in (0, 1):
            i = i2 * 2 + s
            b = base + i
            nxt = 1 - s

            if s == 0:
                pltpu.async_copy(x_hbm.at[b + 1], x_v.at[nxt], sx[nxt])
            else:
                @pl.when(i + 1 < B_PER_W)
                def _():
                    pltpu.async_copy(x_hbm.at[b + 1], x_v.at[nxt], sx[nxt])

            pltpu.make_async_copy(x_hbm.at[b], x_v.at[s], sx[s]).wait()

            @pl.when(i2 > 0)
            def _():
                pltpu.make_async_copy(fwd_v.at[s], out_hbm.at[b - 2],
                                      sf[s]).wait()
                pltpu.make_async_copy(rc_v.at[s], out_hbm.at[B + b - 2],
                                      sr[s]).wait()

            @plsc.parallel_loop(0, CHUNKS, 1, unroll=4)
            def body_c(c):
                xv = x_v[s, pl.ds(c * LANES, LANES)]
                xr = lax.rev(xv, (0,))
                for e in range(E):
                    f = plsc.load_gather(tbl_v, [xv + (e * V)])
                    fwd_v[s, e, pl.ds(c * LANES, LANES)] = f
                    r = plsc.load_gather(tbl_v, [xr + (E * V + e * V)])
                    rc_v[s, e, pl.ds(L - LANES - c * LANES, LANES)] = r

            pltpu.async_copy(fwd_v.at[s], out_hbm.at[b], sf[s])
            pltpu.async_copy(rc_v.at[s], out_hbm.at[B + b], sr[s])
        return carry

    lax.fori_loop(0, B_PER_W // 2, body_i2, 0)

    last = base + B_PER_W - 2
    pltpu.make_async_copy(fwd_v.at[0], out_hbm.at[last], sf[0]).wait()
    pltpu.make_async_copy(rc_v.at[0], out_hbm.at[B + last], sr[0]).wait()
    pltpu.make_async_copy(fwd_v.at[1], out_hbm.at[last + 1], sf[1]).wait()
    pltpu.make_async_copy(rc_v.at[1], out_hbm.at[B + last + 1], sr[1]).wait()


def kernel(x, weight, weight_rc):
    tbl = jnp.concatenate([
        weight.T.reshape(-1),
        weight_rc.T.reshape(-1),
        jnp.zeros((64 - 2 * E * V,), jnp.float32),
    ])
    return _emb_kernel(tbl, x)


# speedup vs baseline: 415.5828x; 1.0007x over previous
import functools

import jax
import jax.numpy as jnp
from jax import lax
from jax.experimental import pallas as pl
from jax.experimental.pallas import tpu as pltpu
from jax.experimental.pallas import tpu_sc as plsc

NUM_CORES = 2
NUM_SUBCORES = 16
LANES = 16
NW = NUM_CORES * NUM_SUBCORES

B = 1024
L = 4096
E = 4
V = 5

B_PER_W = B // NW
CHUNKS = L // LANES

_mesh = plsc.VectorSubcoreMesh(core_axis_name="c", subcore_axis_name="s")


@functools.partial(
    pl.kernel,
    out_type=jax.ShapeDtypeStruct((2 * B, E, L), jnp.float32),
    mesh=_mesh,
    compiler_params=pltpu.CompilerParams(needs_layout_passes=False),
    scratch_types=[
        pltpu.VMEM((64,), jnp.float32),
        pltpu.VMEM((2, L), jnp.int32),
        pltpu.VMEM((2, E, L), jnp.float32),
        pltpu.VMEM((2, E, L), jnp.float32),
        pltpu.SemaphoreType.DMA,
        pltpu.SemaphoreType.DMA,
        pltpu.SemaphoreType.DMA,
        pltpu.SemaphoreType.DMA,
        pltpu.SemaphoreType.DMA,
        pltpu.SemaphoreType.DMA,
    ],
)
def _emb_kernel(tbl_hbm, x_hbm, out_hbm, tbl_v, x_v, fwd_v, rc_v,
                sx0, sx1, sf0, sf1, sr0, sr1):
    wid = lax.axis_index("s") * NUM_CORES + lax.axis_index("c")
    pltpu.sync_copy(tbl_hbm, tbl_v)
    base = wid * B_PER_W
    sx = (sx0, sx1)
    sf = (sf0, sf1)
    sr = (sr0, sr1)

    pltpu.async_copy(x_hbm.at[base], x_v.at[0], sx[0])

    def body_i2(i2, carry):
        for s in (0, 1):
            i = i2 * 2 + s
            b = base + i
            nxt = 1 - s

            if s == 0:
                pltpu.async_copy(x_hbm.at[b + 1], x_v.at[nxt], sx[nxt])
            else:
                @pl.when(i + 1 < B_PER_W)
                def _():
                    pltpu.async_copy(x_hbm.at[b + 1], x_v.at[nxt], sx[nxt])

            pltpu.make_async_copy(x_hbm.at[b], x_v.at[s], sx[s]).wait()

            @pl.when(i2 > 0)
            def _():
                pltpu.make_async_copy(fwd_v.at[s], out_hbm.at[b - 2],
                                      sf[s]).wait()
                pltpu.make_async_copy(rc_v.at[s], out_hbm.at[B + b - 2],
                                      sr[s]).wait()

            @plsc.parallel_loop(0, CHUNKS, 1, unroll=8)
            def body_c(c):
                xv = x_v[s, pl.ds(c * LANES, LANES)]
                xr = lax.rev(xv, (0,))
                for e in range(E):
                    f = plsc.load_gather(tbl_v, [xv + (e * V)])
                    fwd_v[s, e, pl.ds(c * LANES, LANES)] = f
                    r = plsc.load_gather(tbl_v, [xr + (E * V + e * V)])
                    rc_v[s, e, pl.ds(L - LANES - c * LANES, LANES)] = r

            pltpu.async_copy(fwd_v.at[s], out_hbm.at[b], sf[s])
            pltpu.async_copy(rc_v.at[s], out_hbm.at[B + b], sr[s])
        return carry

    lax.fori_loop(0, B_PER_W // 2, body_i2, 0)

    last = base + B_PER_W - 2
    pltpu.make_async_copy(fwd_v.at[0], out_hbm.at[last], sf[0]).wait()
    pltpu.make_async_copy(rc_v.at[0], out_hbm.at[B + last], sr[0]).wait()
    pltpu.make_async_copy(fwd_v.at[1], out_hbm.at[last + 1], sf[1]).wait()
    pltpu.make_async_copy(rc_v.at[1], out_hbm.at[B + last + 1], sr[1]).wait()


def kernel(x, weight, weight_rc):
    tbl = jnp.concatenate([
        weight.T.reshape(-1),
        weight_rc.T.reshape(-1),
        jnp.zeros((64 - 2 * E * V,), jnp.float32),
    ])
    return _emb_kernel(tbl, x)
